# Initial kernel scaffold; baseline (speedup 1.0000x reference)
import jax
import jax.numpy as jnp
from jax.experimental import pallas as pl


def kernel(x, edge_index, W1, b1, W2, b2, Wf1, bf1, Wf2, bf2, Wf3, bf3):
    raise NotImplementedError("write your pallas kernel here")


# speedup vs baseline: 7.9821x; 7.9821x over previous
import functools

import jax
import jax.numpy as jnp
from jax import lax
from jax.experimental import pallas as pl
from jax.experimental.pallas import tpu as pltpu
from jax.experimental.pallas import tpu_sc as plsc

N_NODES = 10000
N_EDGES = 160000
D_IN = 128
D_H = 64

NC = 2
NS = 16
NW = NC * NS
G = 128
GPW = 40
E_PAD = NW * GPW * G
ACC_ROWS = 10112
RPS = ACC_ROWS // NS


def _sc_degree_body(dst2d, ones_hbm, zeros_hbm, out, didx_v, ones_v, acc):
    c = lax.axis_index("c")
    s = lax.axis_index("s")
    w = c * NS + s
    pltpu.sync_copy(dst2d.at[pl.ds(w * GPW, GPW)], didx_v)
    pltpu.sync_copy(ones_hbm, ones_v)
    pltpu.sync_copy(zeros_hbm.at[pl.ds(s * RPS, RPS)], acc.at[pl.ds(s * RPS, RPS)])
    plsc.subcore_barrier()

    def body(j, carry):
        pltpu.sync_copy(ones_v, acc.at[didx_v.at[j]], add=True)
        return carry

    lax.fori_loop(0, GPW, body, 0)
    plsc.subcore_barrier()
    pltpu.sync_copy(acc.at[pl.ds(s * RPS, RPS)], out.at[c, pl.ds(s * RPS, RPS)])


def _sc_aggregate_body(src2d, dst2d, table, zeros_hbm, out,
                       sidx_v, didx_v, rows_v, acc, sem):
    c = lax.axis_index("c")
    s = lax.axis_index("s")
    w = c * NS + s
    pltpu.sync_copy(src2d.at[pl.ds(w * GPW, GPW)], sidx_v)
    pltpu.sync_copy(dst2d.at[pl.ds(w * GPW, GPW)], didx_v)
    pltpu.sync_copy(zeros_hbm.at[pl.ds(s * RPS, RPS)], acc.at[pl.ds(s * RPS, RPS)])
    plsc.subcore_barrier()

    def body(j, carry):
        pltpu.async_copy(table.at[sidx_v.at[j]], rows_v, sem).wait()
        pltpu.sync_copy(rows_v, acc.at[didx_v.at[j]], add=True)
        return carry

    lax.fori_loop(0, GPW, body, 0)
    plsc.subcore_barrier()
    pltpu.sync_copy(acc.at[pl.ds(s * RPS, RPS)], out.at[c, pl.ds(s * RPS, RPS)])


@functools.cache
def _sc_kernels():
    mesh = plsc.VectorSubcoreMesh(core_axis_name="c", subcore_axis_name="s")
    params = pltpu.CompilerParams(use_tc_tiling_on_sc=False)
    degree = pl.kernel(
        _sc_degree_body,
        mesh=mesh,
        compiler_params=params,
        out_type=jax.ShapeDtypeStruct((NC, ACC_ROWS, 16), jnp.float32),
        scratch_types=[
            pltpu.VMEM((GPW, G), jnp.int32),
            pltpu.VMEM((G, 16), jnp.float32),
            pltpu.VMEM_SHARED((ACC_ROWS, 16), jnp.float32),
        ],
    )
    aggregate = pl.kernel(
        _sc_aggregate_body,
        mesh=mesh,
        compiler_params=params,
        out_type=jax.ShapeDtypeStruct((NC, ACC_ROWS, D_H), jnp.float32),
        scratch_types=[
            pltpu.VMEM((GPW, G), jnp.int32),
            pltpu.VMEM((GPW, G), jnp.int32),
            pltpu.VMEM((G, D_H), jnp.float32),
            pltpu.VMEM_SHARED((ACC_ROWS, D_H), jnp.float32),
            pltpu.SemaphoreType.DMA,
        ],
    )
    return degree, aggregate


_TN = 2000


def _tc_scale1_body(deg_ref, x_ref, w1_ref, hs_ref, dis_ref):
    deg = deg_ref[0, :, 0:1] + deg_ref[1, :, 0:1] + 1.0
    dis = lax.rsqrt(deg)
    h = jnp.dot(x_ref[...], w1_ref[...], preferred_element_type=jnp.float32)
    hs_ref[...] = h * dis
    dis_ref[...] = dis


def _tc_combine1_body(raw_ref, hs1_ref, dis_ref, w2_ref, b1_ref, hs2_ref):
    agg = raw_ref[0] + raw_ref[1] + hs1_ref[...]
    emb1 = jnp.maximum(dis_ref[...] * agg + b1_ref[...], 0.0)
    h2 = jnp.dot(emb1, w2_ref[...], preferred_element_type=jnp.float32)
    hs2_ref[...] = h2 * dis_ref[...]


def _tc_combine2_body(raw_ref, hs2_ref, dis_ref, b2_ref, emb2_ref):
    agg = raw_ref[0] + raw_ref[1] + hs2_ref[...]
    emb2_ref[...] = dis_ref[...] * agg + b2_ref[...]


_KT = 16000


def _tc_head_body(flat_ref, wf1_ref, bf1_ref, wf2_ref, bf2_ref, wf3_ref,
                  bf3_ref, q_ref, acc_ref):
    i = pl.program_id(0)

    @pl.when(i == 0)
    def _init():
        acc_ref[...] = jnp.zeros_like(acc_ref)

    acc_ref[...] += jnp.dot(flat_ref[...], wf1_ref[...],
                            preferred_element_type=jnp.float32)

    @pl.when(i == pl.num_programs(0) - 1)
    def _finish():
        z1 = jnp.maximum(acc_ref[...] + bf1_ref[...], 0.0)
        z2 = jnp.maximum(
            jnp.dot(z1, wf2_ref[...], preferred_element_type=jnp.float32)
            + bf2_ref[...], 0.0)
        q_ref[...] = (jnp.dot(z2, wf3_ref[...],
                              preferred_element_type=jnp.float32)
                      + bf3_ref[...])


def _tc_scale1(deg_part, x, W1):
    grid = (N_NODES // _TN,)
    return pl.pallas_call(
        _tc_scale1_body,
        grid=grid,
        in_specs=[
            pl.BlockSpec((NC, _TN, 16), lambda i: (0, i, 0)),
            pl.BlockSpec((_TN, D_IN), lambda i: (i, 0)),
            pl.BlockSpec((D_IN, D_H), lambda i: (0, 0)),
        ],
        out_specs=[
            pl.BlockSpec((_TN, D_H), lambda i: (i, 0)),
            pl.BlockSpec((_TN, 1), lambda i: (i, 0)),
        ],
        out_shape=[
            jax.ShapeDtypeStruct((N_NODES, D_H), jnp.float32),
            jax.ShapeDtypeStruct((N_NODES, 1), jnp.float32),
        ],
    )(deg_part, x, W1)


def _tc_combine1(raw1, hs1, dis, W2, b1):
    grid = (N_NODES // _TN,)
    return pl.pallas_call(
        _tc_combine1_body,
        grid=grid,
        in_specs=[
            pl.BlockSpec((NC, _TN, D_H), lambda i: (0, i, 0)),
            pl.BlockSpec((_TN, D_H), lambda i: (i, 0)),
            pl.BlockSpec((_TN, 1), lambda i: (i, 0)),
            pl.BlockSpec((D_H, D_H), lambda i: (0, 0)),
            pl.BlockSpec((1, D_H), lambda i: (0, 0)),
        ],
        out_specs=pl.BlockSpec((_TN, D_H), lambda i: (i, 0)),
        out_shape=jax.ShapeDtypeStruct((N_NODES, D_H), jnp.float32),
    )(raw1, hs1, dis, W2, b1)


def _tc_combine2(raw2, hs2, dis, b2):
    grid = (N_NODES // _TN,)
    return pl.pallas_call(
        _tc_combine2_body,
        grid=grid,
        in_specs=[
            pl.BlockSpec((NC, _TN, D_H), lambda i: (0, i, 0)),
            pl.BlockSpec((_TN, D_H), lambda i: (i, 0)),
            pl.BlockSpec((_TN, 1), lambda i: (i, 0)),
            pl.BlockSpec((1, D_H), lambda i: (0, 0)),
        ],
        out_specs=pl.BlockSpec((_TN, D_H), lambda i: (i, 0)),
        out_shape=jax.ShapeDtypeStruct((N_NODES, D_H), jnp.float32),
    )(raw2, hs2, dis, b2)


def _tc_head(flat, Wf1, bf1, Wf2, bf2, Wf3, bf3):
    k_total = N_NODES * D_H
    grid = (k_total // _KT,)
    return pl.pallas_call(
        _tc_head_body,
        grid=grid,
        in_specs=[
            pl.BlockSpec((1, _KT), lambda i: (0, i)),
            pl.BlockSpec((_KT, D_H), lambda i: (i, 0)),
            pl.BlockSpec((1, D_H), lambda i: (0, 0)),
            pl.BlockSpec((D_H, D_H), lambda i: (0, 0)),
            pl.BlockSpec((1, D_H), lambda i: (0, 0)),
            pl.BlockSpec((D_H, 1), lambda i: (0, 0)),
            pl.BlockSpec((1, 1), lambda i: (0, 0)),
        ],
        out_specs=pl.BlockSpec((1, 1), lambda i: (0, 0)),
        out_shape=jax.ShapeDtypeStruct((1, 1), jnp.float32),
        scratch_shapes=[pltpu.VMEM((1, D_H), jnp.float32)],
    )(flat, Wf1, bf1, Wf2, bf2, Wf3, bf3)


def kernel(x, edge_index, W1, b1, W2, b2, Wf1, bf1, Wf2, bf2, Wf3, bf3):
    src = edge_index[0]
    dst = edge_index[1]
    pad = E_PAD - N_EDGES
    src2d = jnp.concatenate(
        [src, jnp.zeros((pad,), jnp.int32)]).reshape(E_PAD // G, G)
    dst2d = jnp.concatenate(
        [dst, jnp.full((pad,), N_NODES, jnp.int32)]).reshape(E_PAD // G, G)

    ones16 = jnp.ones((G, 16), jnp.float32)
    zeros_deg = jnp.zeros((ACC_ROWS, 16), jnp.float32)
    zeros_acc = jnp.zeros((ACC_ROWS, D_H), jnp.float32)

    sc_degree, sc_aggregate = _sc_kernels()
    deg_part = sc_degree(dst2d, ones16, zeros_deg)
    hs1, dis = _tc_scale1(deg_part, x, W1)
    raw1 = sc_aggregate(src2d, dst2d, hs1, zeros_acc)
    hs2 = _tc_combine1(raw1, hs1, dis, W2, b1.reshape(1, D_H))
    raw2 = sc_aggregate(src2d, dst2d, hs2, zeros_acc)
    emb2 = _tc_combine2(raw2, hs2, dis, b2.reshape(1, D_H))
    flat = emb2.reshape(1, N_NODES * D_H)
    q = _tc_head(flat, Wf1, bf1.reshape(1, D_H), Wf2, bf2.reshape(1, D_H),
                 Wf3, bf3.reshape(1, 1))
    return q.reshape(1)


# speedup vs baseline: 8.2806x; 1.0374x over previous
import functools

import jax
import jax.numpy as jnp
from jax import lax
from jax.experimental import pallas as pl
from jax.experimental.pallas import tpu as pltpu
from jax.experimental.pallas import tpu_sc as plsc

N_NODES = 10000
N_EDGES = 160000
D_IN = 128
D_H = 64

NC = 2
NS = 16
NW = NC * NS
G = 128
GPW = 40
E_PAD = NW * GPW * G
ACC_ROWS = 10112
RPS = ACC_ROWS // NS


def _sc_degree_body(dst2d, ones_hbm, zeros_hbm, out, didx_v, ones_v, acc):
    c = lax.axis_index("c")
    s = lax.axis_index("s")
    w = c * NS + s
    pltpu.sync_copy(dst2d.at[pl.ds(w * GPW, GPW)], didx_v)
    pltpu.sync_copy(ones_hbm, ones_v)
    pltpu.sync_copy(zeros_hbm.at[pl.ds(s * RPS, RPS)], acc.at[pl.ds(s * RPS, RPS)])
    plsc.subcore_barrier()

    def body(j, carry):
        pltpu.sync_copy(ones_v, acc.at[didx_v.at[j]], add=True)
        return carry

    lax.fori_loop(0, GPW, body, 0)
    plsc.subcore_barrier()
    pltpu.sync_copy(acc.at[pl.ds(s * RPS, RPS)], out.at[c, pl.ds(s * RPS, RPS)])


KB = 4
NB = GPW // KB


def _sc_aggregate_body(src2d, dst2d, table, zeros_hbm, out,
                       sidx_v, didx_v, rows_v, sem_g, sem_s, acc):
    c = lax.axis_index("c")
    s = lax.axis_index("s")
    w = c * NS + s
    pltpu.sync_copy(src2d.at[pl.ds(w * GPW, GPW)], sidx_v)
    pltpu.sync_copy(dst2d.at[pl.ds(w * GPW, GPW)], didx_v)
    pltpu.sync_copy(zeros_hbm.at[pl.ds(s * RPS, RPS)], acc.at[pl.ds(s * RPS, RPS)])
    plsc.subcore_barrier()

    def fire_gather(buf, i, g):
        pltpu.async_copy(table.at[sidx_v.at[g]], rows_v.at[buf, i],
                         sem_g.at[buf, i])

    def wait_gather(buf, i, g):
        pltpu.make_async_copy(table.at[sidx_v.at[g]], rows_v.at[buf, i],
                              sem_g.at[buf, i]).wait()

    def run_block(p, buf, refire):
        descs = []
        for i in range(KB):
            g = p * KB + i
            wait_gather(buf, i, g)
            descs.append(pltpu.async_copy(rows_v.at[buf, i],
                                          acc.at[didx_v.at[g]],
                                          sem_s.at[buf, i], add=True))
        for i in range(KB):
            descs[i].wait()
            if refire:
                fire_gather(buf, i, (p + 2) * KB + i)

    for buf in range(2):
        for i in range(KB):
            fire_gather(buf, i, buf * KB + i)

    def body(q, carry):
        run_block(2 * q, 0, True)
        run_block(2 * q + 1, 1, True)
        return carry

    lax.fori_loop(0, NB // 2 - 1, body, 0)
    run_block(NB - 2, 0, False)
    run_block(NB - 1, 1, False)

    plsc.subcore_barrier()
    pltpu.sync_copy(acc.at[pl.ds(s * RPS, RPS)], out.at[c, pl.ds(s * RPS, RPS)])


@functools.cache
def _sc_kernels():
    mesh = plsc.VectorSubcoreMesh(core_axis_name="c", subcore_axis_name="s")
    params = pltpu.CompilerParams(use_tc_tiling_on_sc=False)
    degree = pl.kernel(
        _sc_degree_body,
        mesh=mesh,
        compiler_params=params,
        out_type=jax.ShapeDtypeStruct((NC, ACC_ROWS, 16), jnp.float32),
        scratch_types=[
            pltpu.VMEM((GPW, G), jnp.int32),
            pltpu.VMEM((G, 16), jnp.float32),
            pltpu.VMEM_SHARED((ACC_ROWS, 16), jnp.float32),
        ],
    )
    aggregate = pl.kernel(
        _sc_aggregate_body,
        mesh=mesh,
        compiler_params=params,
        out_type=jax.ShapeDtypeStruct((NC, ACC_ROWS, D_H), jnp.float32),
        scratch_types=[
            pltpu.VMEM((GPW, G), jnp.int32),
            pltpu.VMEM((GPW, G), jnp.int32),
            pltpu.VMEM((2, KB, G, D_H), jnp.float32),
            pltpu.SemaphoreType.DMA((2, KB)),
            pltpu.SemaphoreType.DMA((2, KB)),
            pltpu.VMEM_SHARED((ACC_ROWS, D_H), jnp.float32),
        ],
    )
    return degree, aggregate


_TN = 2000


def _tc_scale1_body(deg_ref, x_ref, w1_ref, hs_ref, dis_ref):
    deg = deg_ref[0, :, 0:1] + deg_ref[1, :, 0:1] + 1.0
    dis = lax.rsqrt(deg)
    h = jnp.dot(x_ref[...], w1_ref[...], preferred_element_type=jnp.float32)
    hs_ref[...] = h * dis
    dis_ref[...] = dis


def _tc_combine1_body(raw_ref, hs1_ref, dis_ref, w2_ref, b1_ref, hs2_ref):
    agg = raw_ref[0] + raw_ref[1] + hs1_ref[...]
    emb1 = jnp.maximum(dis_ref[...] * agg + b1_ref[...], 0.0)
    h2 = jnp.dot(emb1, w2_ref[...], preferred_element_type=jnp.float32)
    hs2_ref[...] = h2 * dis_ref[...]


def _tc_combine2_body(raw_ref, hs2_ref, dis_ref, b2_ref, emb2_ref):
    agg = raw_ref[0] + raw_ref[1] + hs2_ref[...]
    emb2_ref[...] = dis_ref[...] * agg + b2_ref[...]


_KT = 16000


def _tc_head_body(flat_ref, wf1_ref, bf1_ref, wf2_ref, bf2_ref, wf3_ref,
                  bf3_ref, q_ref, acc_ref):
    i = pl.program_id(0)

    @pl.when(i == 0)
    def _init():
        acc_ref[...] = jnp.zeros_like(acc_ref)

    acc_ref[...] += jnp.dot(flat_ref[...], wf1_ref[...],
                            preferred_element_type=jnp.float32)

    @pl.when(i == pl.num_programs(0) - 1)
    def _finish():
        z1 = jnp.maximum(acc_ref[...] + bf1_ref[...], 0.0)
        z2 = jnp.maximum(
            jnp.dot(z1, wf2_ref[...], preferred_element_type=jnp.float32)
            + bf2_ref[...], 0.0)
        q_ref[...] = (jnp.dot(z2, wf3_ref[...],
                              preferred_element_type=jnp.float32)
                      + bf3_ref[...])


def _tc_scale1(deg_part, x, W1):
    grid = (N_NODES // _TN,)
    return pl.pallas_call(
        _tc_scale1_body,
        grid=grid,
        in_specs=[
            pl.BlockSpec((NC, _TN, 16), lambda i: (0, i, 0)),
            pl.BlockSpec((_TN, D_IN), lambda i: (i, 0)),
            pl.BlockSpec((D_IN, D_H), lambda i: (0, 0)),
        ],
        out_specs=[
            pl.BlockSpec((_TN, D_H), lambda i: (i, 0)),
            pl.BlockSpec((_TN, 1), lambda i: (i, 0)),
        ],
        out_shape=[
            jax.ShapeDtypeStruct((N_NODES, D_H), jnp.float32),
            jax.ShapeDtypeStruct((N_NODES, 1), jnp.float32),
        ],
    )(deg_part, x, W1)


def _tc_combine1(raw1, hs1, dis, W2, b1):
    grid = (N_NODES // _TN,)
    return pl.pallas_call(
        _tc_combine1_body,
        grid=grid,
        in_specs=[
            pl.BlockSpec((NC, _TN, D_H), lambda i: (0, i, 0)),
            pl.BlockSpec((_TN, D_H), lambda i: (i, 0)),
            pl.BlockSpec((_TN, 1), lambda i: (i, 0)),
            pl.BlockSpec((D_H, D_H), lambda i: (0, 0)),
            pl.BlockSpec((1, D_H), lambda i: (0, 0)),
        ],
        out_specs=pl.BlockSpec((_TN, D_H), lambda i: (i, 0)),
        out_shape=jax.ShapeDtypeStruct((N_NODES, D_H), jnp.float32),
    )(raw1, hs1, dis, W2, b1)


def _tc_combine2(raw2, hs2, dis, b2):
    grid = (N_NODES // _TN,)
    return pl.pallas_call(
        _tc_combine2_body,
        grid=grid,
        in_specs=[
            pl.BlockSpec((NC, _TN, D_H), lambda i: (0, i, 0)),
            pl.BlockSpec((_TN, D_H), lambda i: (i, 0)),
            pl.BlockSpec((_TN, 1), lambda i: (i, 0)),
            pl.BlockSpec((1, D_H), lambda i: (0, 0)),
        ],
        out_specs=pl.BlockSpec((_TN, D_H), lambda i: (i, 0)),
        out_shape=jax.ShapeDtypeStruct((N_NODES, D_H), jnp.float32),
    )(raw2, hs2, dis, b2)


def _tc_head(flat, Wf1, bf1, Wf2, bf2, Wf3, bf3):
    k_total = N_NODES * D_H
    grid = (k_total // _KT,)
    return pl.pallas_call(
        _tc_head_body,
        grid=grid,
        in_specs=[
            pl.BlockSpec((1, _KT), lambda i: (0, i)),
            pl.BlockSpec((_KT, D_H), lambda i: (i, 0)),
            pl.BlockSpec((1, D_H), lambda i: (0, 0)),
            pl.BlockSpec((D_H, D_H), lambda i: (0, 0)),
            pl.BlockSpec((1, D_H), lambda i: (0, 0)),
            pl.BlockSpec((D_H, 1), lambda i: (0, 0)),
            pl.BlockSpec((1, 1), lambda i: (0, 0)),
        ],
        out_specs=pl.BlockSpec((1, 1), lambda i: (0, 0)),
        out_shape=jax.ShapeDtypeStruct((1, 1), jnp.float32),
        scratch_shapes=[pltpu.VMEM((1, D_H), jnp.float32)],
    )(flat, Wf1, bf1, Wf2, bf2, Wf3, bf3)


def kernel(x, edge_index, W1, b1, W2, b2, Wf1, bf1, Wf2, bf2, Wf3, bf3):
    src = edge_index[0]
    dst = edge_index[1]
    pad = E_PAD - N_EDGES
    src2d = jnp.concatenate(
        [src, jnp.zeros((pad,), jnp.int32)]).reshape(E_PAD // G, G)
    dst2d = jnp.concatenate(
        [dst, jnp.full((pad,), N_NODES, jnp.int32)]).reshape(E_PAD // G, G)

    ones16 = jnp.ones((G, 16), jnp.float32)
    zeros_deg = jnp.zeros((ACC_ROWS, 16), jnp.float32)
    zeros_acc = jnp.zeros((ACC_ROWS, D_H), jnp.float32)

    sc_degree, sc_aggregate = _sc_kernels()
    deg_part = sc_degree(dst2d, ones16, zeros_deg)
    hs1, dis = _tc_scale1(deg_part, x, W1)
    raw1 = sc_aggregate(src2d, dst2d, hs1, zeros_acc)
    hs2 = _tc_combine1(raw1, hs1, dis, W2, b1.reshape(1, D_H))
    raw2 = sc_aggregate(src2d, dst2d, hs2, zeros_acc)
    emb2 = _tc_combine2(raw2, hs2, dis, b2.reshape(1, D_H))
    flat = emb2.reshape(1, N_NODES * D_H)
    q = _tc_head(flat, Wf1, bf1.reshape(1, D_H), Wf2, bf2.reshape(1, D_H),
                 Wf3, bf3.reshape(1, 1))
    return q.reshape(1)
